# Initial kernel scaffold; baseline (speedup 1.0000x reference)
import functools

import jax
import jax.numpy as jnp
from jax import lax
from jax.experimental import pallas as pl
from jax.experimental.pallas import tpu as pltpu
from jax.experimental.pallas import tpu_sc as plsc

VOCAB = 1000000
MAX_LEN = 200
D = 64
EPS = 1e-5

NC = 2
NS = 16
NW = NC * NS
NTOK = 4096 * 200
TOK_PER_W = NTOK // NW
CH = 128
STEPS = TOK_PER_W // CH


def _rsqrt_vec(x):
    i = lax.bitcast_convert_type(x, jnp.int32)
    i = jnp.int32(0x5F3759DF) - lax.shift_right_logical(i, 1)
    y = lax.bitcast_convert_type(i, jnp.float32)
    for _ in range(3):
        y = y * (jnp.float32(1.5) - jnp.float32(0.5) * x * y * y)
    return y


@functools.partial(
    pl.kernel,
    mesh=plsc.VectorSubcoreMesh(core_axis_name="c", subcore_axis_name="s"),
    out_type=jax.ShapeDtypeStruct((NTOK, D), jnp.float32),
    scratch_types=[
        pltpu.VMEM((TOK_PER_W,), jnp.int32),
        pltpu.VMEM((CH, D), jnp.float32),
        pltpu.VMEM((MAX_LEN, D), jnp.float32),
        pltpu.SemaphoreType.DMA,
    ],
)
def _emb_body(x_hbm, tok_hbm, pos_hbm, out_hbm, idx_v, rows_v, pos_v, sem):
    wid = lax.axis_index("s") * NC + lax.axis_index("c")
    base = wid * TOK_PER_W
    pltpu.sync_copy(pos_hbm, pos_v)
    pltpu.sync_copy(x_hbm.at[pl.ds(base, TOK_PER_W)], idx_v)

    def step(g, carry):
        off = g * CH
        pltpu.async_copy(
            tok_hbm.at[idx_v.at[pl.ds(off, CH)]], rows_v, sem
        ).wait()

        def token(r, c):
            p = lax.rem(off + r, MAX_LEN)
            v = [
                rows_v[r, pl.ds(16 * k, 16)] + pos_v[p, pl.ds(16 * k, 16)]
                for k in range(4)
            ]
            total = jnp.sum((v[0] + v[1]) + (v[2] + v[3]))
            mean = total * jnp.float32(1.0 / 64.0)
            d = [vk - mean for vk in v]
            sq = [dk * dk for dk in d]
            var = jnp.sum((sq[0] + sq[1]) + (sq[2] + sq[3])) * jnp.float32(
                1.0 / 64.0
            )
            rs = _rsqrt_vec(jnp.broadcast_to(var + jnp.float32(EPS), (16,)))
            for k in range(4):
                rows_v[r, pl.ds(16 * k, 16)] = d[k] * rs
            return c

        lax.fori_loop(0, CH, token, 0)
        pltpu.sync_copy(rows_v, out_hbm.at[pl.ds(base + off, CH)])
        return carry

    lax.fori_loop(0, STEPS, step, 0)


def kernel(x, tok_table, pos_table):
    b, s = x.shape
    xf = x.reshape(-1).astype(jnp.int32)
    out = _emb_body(xf, tok_table, pos_table)
    return out.reshape(b, s, D)


# speedup vs baseline: 1.3667x; 1.3667x over previous
import functools

import jax
import jax.numpy as jnp
import numpy as np
from jax import lax
from jax.experimental import pallas as pl
from jax.experimental.pallas import tpu as pltpu
from jax.experimental.pallas import tpu_sc as plsc

VOCAB = 1000000
MAX_LEN = 200
D = 64
EPS = 1e-5

NC = 2
NS = 16
NW = NC * NS
NTOK = 4096 * 200
TOK_PER_W = NTOK // NW
CH = 128
STEPS = TOK_PER_W // CH


def _shuffle(s, perm):
    return lax.gather(
        s,
        perm,
        dimension_numbers=lax.GatherDimensionNumbers(
            offset_dims=(), collapsed_slice_dims=(0,), start_index_map=(0,)
        ),
        slice_sizes=(1,),
        mode=lax.GatherScatterMode.PROMISE_IN_BOUNDS,
    )


def _xsum(s):
    lanes = lax.iota(jnp.int32, 16)
    for k in (8, 4, 2, 1):
        perm = jnp.reshape(lanes ^ k, (16, 1))
        s = s + _shuffle(s, perm)
    return s


def _rsqrt_vec(x):
    i = lax.bitcast_convert_type(x, jnp.int32)
    i = jnp.int32(0x5F3759DF) - lax.shift_right_logical(i, 1)
    y = lax.bitcast_convert_type(i, jnp.float32)
    for _ in range(3):
        y = y * (jnp.float32(1.5) - jnp.float32(0.5) * x * y * y)
    return y


@functools.partial(
    pl.kernel,
    mesh=plsc.VectorSubcoreMesh(core_axis_name="c", subcore_axis_name="s"),
    compiler_params=pltpu.CompilerParams(use_tc_tiling_on_sc=False),
    out_type=jax.ShapeDtypeStruct((NTOK, D), jnp.float32),
    scratch_types=[
        pltpu.VMEM((TOK_PER_W,), jnp.int32),
        pltpu.VMEM((CH, D), jnp.float32),
        pltpu.VMEM((MAX_LEN, D), jnp.float32),
        pltpu.SemaphoreType.DMA,
    ],
)
def _emb_body(x_hbm, tok_hbm, pos_hbm, out_hbm, idx_v, rows_v, pos_v, sem):
    wid = lax.axis_index("s") * NC + lax.axis_index("c")
    base = wid * TOK_PER_W
    pltpu.sync_copy(pos_hbm, pos_v)
    pltpu.sync_copy(x_hbm.at[pl.ds(base, TOK_PER_W)], idx_v)

    def step(g, carry):
        off = g * CH
        pltpu.async_copy(
            tok_hbm.at[idx_v.at[pl.ds(off, CH)]], rows_v, sem
        ).wait()

        def token(r, c):
            p = lax.rem(off + r, MAX_LEN)
            v = [
                rows_v[r, pl.ds(16 * k, 16)] + pos_v[p, pl.ds(16 * k, 16)]
                for k in range(4)
            ]
            mean = _xsum((v[0] + v[1]) + (v[2] + v[3])) * jnp.float32(1.0 / 64.0)
            d = [vk - mean for vk in v]
            sq = [dk * dk for dk in d]
            var = _xsum((sq[0] + sq[1]) + (sq[2] + sq[3])) * jnp.float32(
                1.0 / 64.0
            )
            rs = _rsqrt_vec(var + jnp.float32(EPS))
            for k in range(4):
                rows_v[r, pl.ds(16 * k, 16)] = d[k] * rs
            return c

        lax.fori_loop(0, CH, token, 0)
        pltpu.sync_copy(rows_v, out_hbm.at[pl.ds(base + off, CH)])
        return carry

    lax.fori_loop(0, STEPS, step, 0)


def kernel(x, tok_table, pos_table):
    b, s = x.shape
    xf = x.reshape(-1).astype(jnp.int32)
    out = _emb_body(xf, tok_table, pos_table)
    return out.reshape(b, s, D)


# speedup vs baseline: 1.4249x; 1.0426x over previous
import functools

import jax
import jax.numpy as jnp
from jax import lax
from jax.experimental import pallas as pl
from jax.experimental.pallas import tpu as pltpu
from jax.experimental.pallas import tpu_sc as plsc

VOCAB = 1000000
MAX_LEN = 200
D = 64
EPS = 1e-5

NC = 2
NS = 16
NW = NC * NS
NTOK = 4096 * 200
TOK_PER_W = NTOK // NW
CH = 128
STEPS = TOK_PER_W // CH
NBUF = 2


def _shuffle(s, perm):
    return lax.gather(
        s,
        perm,
        dimension_numbers=lax.GatherDimensionNumbers(
            offset_dims=(), collapsed_slice_dims=(0,), start_index_map=(0,)
        ),
        slice_sizes=(1,),
        mode=lax.GatherScatterMode.PROMISE_IN_BOUNDS,
    )


def _xsum(s):
    lanes = lax.iota(jnp.int32, 16)
    for k in (8, 4, 2, 1):
        perm = jnp.reshape(lanes ^ k, (16, 1))
        s = s + _shuffle(s, perm)
    return s


def _rsqrt_vec(x):
    i = lax.bitcast_convert_type(x, jnp.int32)
    i = jnp.int32(0x5F3759DF) - lax.shift_right_logical(i, 1)
    y = lax.bitcast_convert_type(i, jnp.float32)
    for _ in range(3):
        y = y * (jnp.float32(1.5) - jnp.float32(0.5) * x * y * y)
    return y


@functools.partial(
    pl.kernel,
    mesh=plsc.VectorSubcoreMesh(core_axis_name="c", subcore_axis_name="s"),
    compiler_params=pltpu.CompilerParams(use_tc_tiling_on_sc=False),
    out_type=jax.ShapeDtypeStruct((NTOK // 2, 128), jnp.float32),
    scratch_types=(
        [pltpu.VMEM((TOK_PER_W // 16, 16), jnp.int32)]
        + [pltpu.VMEM((CH,), jnp.int32)] * NBUF
        + [pltpu.VMEM((CH, 128), jnp.float32)] * NBUF
        + [pltpu.VMEM((CH // 2, 128), jnp.float32)] * NBUF
        + [pltpu.VMEM((MAX_LEN // 2, 128), jnp.float32)]
        + [pltpu.SemaphoreType.DMA] * (2 * NBUF)
    ),
)
def _emb_body(x_hbm, tok_hbm, pos_hbm, out_hbm, idx_v, *scratch):
    idx2 = scratch[0:NBUF]
    in_b = scratch[NBUF:2 * NBUF]
    out_b = scratch[2 * NBUF:3 * NBUF]
    pos_v = scratch[3 * NBUF]
    sem_g = scratch[3 * NBUF + 1:3 * NBUF + 1 + NBUF]
    sem_w = scratch[3 * NBUF + 1 + NBUF:3 * NBUF + 1 + 2 * NBUF]

    wid = lax.axis_index("s") * NC + lax.axis_index("c")
    base = pl.multiple_of(wid * TOK_PER_W, TOK_PER_W)
    pltpu.sync_copy(pos_hbm, pos_v)
    pltpu.sync_copy(x_hbm.at[pl.ds(base // 16, TOK_PER_W // 16)], idx_v)

    def gather(g, b):
        row0 = g * (CH // 16)
        for c in range(CH // 16):
            idx2[b][pl.ds(16 * c, 16)] = lax.shift_right_logical(
                idx_v[row0 + c], 1
            )
        pltpu.async_copy(tok_hbm.at[idx2[b]], in_b[b], sem_g[b])

    for b in range(NBUF):
        gather(b, b)

    @pl.loop(0, STEPS, step=NBUF)
    def outer(g0):
        for b in range(NBUF):
            g = g0 + b
            off = g * CH
            pltpu.make_async_copy(
                tok_hbm.at[idx2[b]], in_b[b], sem_g[b]
            ).wait()
            @pl.when(g0 > 0)
            def _():
                pltpu.make_async_copy(
                    out_b[b],
                    out_hbm.at[
                        pl.ds(
                            pl.multiple_of(
                                (base + off - NBUF * CH) // 2, CH // 2
                            ),
                            CH // 2,
                        )
                    ],
                    sem_w[b],
                ).wait()

            off_mod = lax.rem(off, MAX_LEN)
            lanes = lax.iota(jnp.int32, 16)

            @plsc.parallel_loop(0, CH // 16, unroll=1)
            def group(gi):
                pv = lax.bitwise_and(idx_v[off // 16 + gi], 1)

                def ln(j):
                    row = 16 * gi + j
                    p = off_mod + row
                    p = jnp.where(p >= MAX_LEN, p - MAX_LEN, p)
                    ph = lax.shift_right_logical(p, 1)
                    pcol = (j % 2) * D
                    parf = _shuffle(pv, jnp.reshape(lanes * 0 + j, (16, 1)))
                    parf = parf.astype(jnp.float32)
                    v = []
                    for k in range(4):
                        h0 = in_b[b][row, pl.ds(16 * k, 16)]
                        h1 = in_b[b][row, pl.ds(D + 16 * k, 16)]
                        t = h0 + parf * (h1 - h0)
                        v.append(t + pos_v[ph, pl.ds(pcol + 16 * k, 16)])
                    mean = _xsum((v[0] + v[1]) + (v[2] + v[3])) * jnp.float32(
                        1.0 / 64.0
                    )
                    d = [vk - mean for vk in v]
                    sq = [dk * dk for dk in d]
                    var = _xsum(
                        (sq[0] + sq[1]) + (sq[2] + sq[3])
                    ) * jnp.float32(1.0 / 64.0)
                    rs = _rsqrt_vec(var + jnp.float32(EPS))
                    for k in range(4):
                        out_b[b][8 * gi + j // 2, pl.ds(pcol + 16 * k, 16)] = (
                            d[k] * rs
                        )

                for j in range(16):
                    ln(j)

            @pl.when(g + NBUF < STEPS)
            def _():
                gather(g + NBUF, b)

            pltpu.async_copy(
                out_b[b],
                out_hbm.at[pl.ds(pl.multiple_of((base + off) // 2, CH // 2), CH // 2)],
                sem_w[b],
            )

    for b in range(NBUF):
        off = (STEPS - NBUF + b) * CH
        pltpu.make_async_copy(
            out_b[b],
            out_hbm.at[pl.ds(pl.multiple_of((base + off) // 2, CH // 2), CH // 2)],
            sem_w[b],
        ).wait()


def kernel(x, tok_table, pos_table):
    b, s = x.shape
    xf = x.reshape(NTOK // 16, 16).astype(jnp.int32)
    t2 = tok_table.reshape(VOCAB // 2, 128)
    p2 = pos_table.reshape(MAX_LEN // 2, 128)
    out = _emb_body(xf, t2, p2)
    return out.reshape(b, s, D)


# speedup vs baseline: 2.7744x; 1.9470x over previous
import functools

import jax
import jax.numpy as jnp
from jax import lax
from jax.experimental import pallas as pl
from jax.experimental.pallas import tpu as pltpu
from jax.experimental.pallas import tpu_sc as plsc

VOCAB = 1000000
MAX_LEN = 200
D = 64
EPS = 1e-5

NC = 2
NS = 16
NW = NC * NS
BATCH = 4096
NTOK = BATCH * MAX_LEN
TOK_PER_W = NTOK // NW
CH = 128
STEPS = TOK_PER_W // CH
NBUF = 4


def _shuffle(s, perm):
    return lax.gather(
        s,
        perm,
        dimension_numbers=lax.GatherDimensionNumbers(
            offset_dims=(), collapsed_slice_dims=(0,), start_index_map=(0,)
        ),
        slice_sizes=(1,),
        mode=lax.GatherScatterMode.PROMISE_IN_BOUNDS,
    )


def _xsum(s):
    lanes = lax.iota(jnp.int32, 16)
    for k in (8, 4, 2, 1):
        perm = jnp.reshape(lanes ^ k, (16, 1))
        s = s + _shuffle(s, perm)
    return s


def _rsqrt_vec(x):
    i = lax.bitcast_convert_type(x, jnp.int32)
    i = jnp.int32(0x5F3759DF) - lax.shift_right_logical(i, 1)
    y = lax.bitcast_convert_type(i, jnp.float32)
    for _ in range(3):
        y = y * (jnp.float32(1.5) - jnp.float32(0.5) * x * y * y)
    return y


@functools.partial(
    pl.kernel,
    mesh=plsc.VectorSubcoreMesh(core_axis_name="c", subcore_axis_name="s"),
    compiler_params=pltpu.CompilerParams(use_tc_tiling_on_sc=False),
    out_type=jax.ShapeDtypeStruct((NTOK, D), jnp.float32),
    scratch_types=(
        [pltpu.VMEM((TOK_PER_W,), jnp.int32)]
        + [pltpu.VMEM((CH, D), jnp.float32)] * NBUF
        + [pltpu.VMEM((CH, D), jnp.float32)] * NBUF
        + [pltpu.VMEM((MAX_LEN, D), jnp.float32)]
        + [pltpu.SemaphoreType.DMA] * (2 * NBUF)
    ),
)
def _emb_body(x_hbm, tok_hbm, pos_hbm, out_hbm, idx_v, *scratch):
    in_b = scratch[0:NBUF]
    out_b = scratch[NBUF:2 * NBUF]
    pos_v = scratch[2 * NBUF]
    sem_g = scratch[2 * NBUF + 1:2 * NBUF + 1 + NBUF]
    sem_w = scratch[2 * NBUF + 1 + NBUF:2 * NBUF + 1 + 2 * NBUF]

    wid = lax.axis_index("s") * NC + lax.axis_index("c")
    base = pl.multiple_of(wid * TOK_PER_W, TOK_PER_W)
    pltpu.sync_copy(pos_hbm, pos_v)
    pltpu.sync_copy(x_hbm.at[pl.ds(base, TOK_PER_W)], idx_v)

    def gather(g, b):
        pltpu.async_copy(
            tok_hbm.at[idx_v.at[pl.ds(g * CH, CH)]], in_b[b], sem_g[b]
        )

    for b in range(NBUF):
        gather(b, b)

    @pl.loop(0, STEPS, step=NBUF)
    def outer(g0):
        for b in range(NBUF):
            g = g0 + b
            off = g * CH
            pltpu.make_async_copy(
                tok_hbm.at[idx_v.at[pl.ds(off, CH)]], in_b[b], sem_g[b]
            ).wait()
            @pl.when(g0 > 0)
            def _():
                pltpu.make_async_copy(
                    out_b[b], out_hbm.at[pl.ds(base + off - NBUF * CH, CH)],
                    sem_w[b],
                ).wait()

            s_pos = (base + off) // BATCH
            pr = [pos_v[s_pos, pl.ds(16 * k, 16)] for k in range(4)]

            @plsc.parallel_loop(0, CH, unroll=4)
            def token(r):
                v = [
                    in_b[b][r, pl.ds(16 * k, 16)] + pr[k] for k in range(4)
                ]
                mean = _xsum((v[0] + v[1]) + (v[2] + v[3])) * jnp.float32(
                    1.0 / 64.0
                )
                d = [vk - mean for vk in v]
                sq = [dk * dk for dk in d]
                var = _xsum((sq[0] + sq[1]) + (sq[2] + sq[3])) * jnp.float32(
                    1.0 / 64.0
                )
                rs = _rsqrt_vec(var + jnp.float32(EPS))
                for k in range(4):
                    out_b[b][r, pl.ds(16 * k, 16)] = d[k] * rs

            @pl.when(g + NBUF < STEPS)
            def _():
                gather(g + NBUF, b)

            pltpu.async_copy(
                out_b[b], out_hbm.at[pl.ds(base + off, CH)], sem_w[b]
            )

    for b in range(NBUF):
        off = (STEPS - NBUF + b) * CH
        pltpu.make_async_copy(
            out_b[b], out_hbm.at[pl.ds(base + off, CH)], sem_w[b]
        ).wait()


def kernel(x, tok_table, pos_table):
    b, s = x.shape
    xs = x.T.reshape(-1).astype(jnp.int32)
    out = _emb_body(xs, tok_table, pos_table)
    return out.reshape(s, b, D).transpose(1, 0, 2)
